# Initial kernel scaffold; baseline (speedup 1.0000x reference)
import jax
import jax.numpy as jnp
from jax.experimental import pallas as pl


def kernel(input, seed):
    raise NotImplementedError("write your pallas kernel here")


# speedup vs baseline: 3.8079x; 3.8079x over previous
import functools

import jax
import jax.numpy as jnp
from jax.experimental import pallas as pl
from jax.experimental.pallas import tpu as pltpu

_TOPK = 128
_N_HEADS = 12
_CHUNK = 1024


def _fused_body(x_ref, q_ref, o_ref, xbf_ref, sc_ref, *, topk, scale,
                n_chunks, chunk):
    c = pl.program_id(1)
    xc = x_ref[0]
    q = q_ref[0]
    sc_ref[:, pl.ds(c * chunk, chunk)] = jax.lax.dot_general(
        q.astype(jnp.bfloat16), xc.astype(jnp.bfloat16),
        (((1,), (1,)), ((), ())),
        preferred_element_type=jnp.float32)
    xbf_ref[pl.ds(c * chunk, chunk), :] = xc.astype(jnp.bfloat16)

    @pl.when(c == n_chunks - 1)
    def _finish():
        scores = sc_ref[...]
        bits = jax.lax.bitcast_convert_type(scores, jnp.int32)
        sortable = bits ^ ((bits >> 31) & jnp.int32(0x7FFFFFFF))

        n_nonneg = jnp.sum((sortable >= 0).astype(jnp.int32), axis=1,
                           keepdims=True)
        int_min = jnp.int32(-(2 ** 31))
        prefix = jnp.where(n_nonneg >= topk, jnp.int32(0), int_min)

        def bit_step(i, pfx):
            bit = jnp.left_shift(jnp.int32(1), jnp.int32(30) - i)
            cand = pfx + bit
            cnt = jnp.sum((sortable >= cand).astype(jnp.int32), axis=1,
                          keepdims=True)
            return jnp.where(cnt >= topk, cand, pfx)

        thresh = jax.lax.fori_loop(0, 31, bit_step, prefix)

        mask = sortable >= thresh
        rmax = jnp.max(scores, axis=1, keepdims=True)
        w = jnp.where(mask, jnp.exp((scores - rmax) * scale), 0.0)
        z = jnp.sum(w, axis=1, keepdims=True)

        out = jax.lax.dot_general(
            w.astype(jnp.bfloat16), xbf_ref[...], (((1,), (0,)), ((), ())),
            preferred_element_type=jnp.float32)
        o_ref[0] = out / z


def kernel(input, seed):
    B, N, H = input.shape
    S = seed.shape[1]
    chunk = min(_CHUNK, N)
    n_chunks = N // chunk
    assert N % chunk == 0
    body = functools.partial(
        _fused_body, topk=min(_TOPK, N), scale=_N_HEADS ** -0.5,
        n_chunks=n_chunks, chunk=chunk)
    return pl.pallas_call(
        body,
        grid=(B, n_chunks),
        in_specs=[
            pl.BlockSpec((1, chunk, H), lambda b, c: (b, c, 0)),
            pl.BlockSpec((1, S, H), lambda b, c: (0, 0, 0)),
        ],
        out_specs=pl.BlockSpec((1, S, H), lambda b, c: (b, 0, 0)),
        out_shape=jax.ShapeDtypeStruct((B, S, H), jnp.float32),
        scratch_shapes=[
            pltpu.VMEM((N, H), jnp.bfloat16),
            pltpu.VMEM((S, N), jnp.float32),
        ],
    )(input, seed)


# speedup vs baseline: 3.8528x; 1.0118x over previous
import functools

import jax
import jax.numpy as jnp
from jax.experimental import pallas as pl
from jax.experimental.pallas import tpu as pltpu

_TOPK = 128
_N_HEADS = 12
_N_CHUNKS = 8


def _radix_span(sortable, prefix, topk, lo, hi):
    def bit_step(i, pfx):
        bit = jnp.left_shift(jnp.int32(1), jnp.int32(30) - i)
        cand = pfx + bit
        cnt = jnp.sum((sortable >= cand).astype(jnp.int32), axis=1,
                      keepdims=True)
        return jnp.where(cnt >= topk, cand, pfx)

    return jax.lax.fori_loop(lo, hi, bit_step, prefix)


def _body(x_ref, q_ref, o_ref, xbf_ref, sc_ref, st_ref, z_ref, *,
          topk, scale, chunk, n_batches):
    vb = pl.program_id(0)
    c = pl.program_id(1)
    cur = vb % 2
    prev = (vb + 1) % 2
    S = q_ref.shape[1]

    @pl.when(vb < n_batches)
    def _chunk():
        xc = x_ref[0]
        q = q_ref[0]
        sc_ref[cur, :, pl.ds(c * chunk, chunk)] = jax.lax.dot_general(
            q.astype(jnp.bfloat16), xc.astype(jnp.bfloat16),
            (((1,), (1,)), ((), ())),
            preferred_element_type=jnp.float32)
        xbf_ref[cur, pl.ds(c * chunk, chunk), :] = xc.astype(jnp.bfloat16)

    p2 = vb > 0
    n = sc_ref.shape[2]

    @pl.when(p2 & (c == 0))
    def _radix0():
        sortable = _sortable(sc_ref[prev])
        n_nonneg = jnp.sum((sortable >= 0).astype(jnp.int32), axis=1,
                           keepdims=True)
        prefix = jnp.where(n_nonneg >= topk, jnp.int32(0),
                           jnp.int32(-(2 ** 31)))
        prefix = _radix_span(sortable, prefix, topk, 0, 10)
        st_ref[...] = jnp.broadcast_to(prefix, st_ref.shape)

    @pl.when(p2 & (c == 1))
    def _radix1():
        sortable = _sortable(sc_ref[prev])
        prefix = _radix_span(sortable, st_ref[:, :1], topk, 10, 20)
        st_ref[...] = jnp.broadcast_to(prefix, st_ref.shape)

    @pl.when(p2 & (c == 2))
    def _radix2():
        sortable = _sortable(sc_ref[prev])
        prefix = _radix_span(sortable, st_ref[:, :1], topk, 20, 31)
        st_ref[...] = jnp.broadcast_to(prefix, st_ref.shape)

    @pl.when(p2 & (c == 3))
    def _weights():
        scores = sc_ref[prev]
        sortable = _sortable(scores)
        mask = sortable >= st_ref[:, :1]
        rmax = jnp.max(scores, axis=1, keepdims=True)
        w = jnp.where(mask, jnp.exp((scores - rmax) * scale), 0.0)
        z = jnp.sum(w, axis=1, keepdims=True)
        sc_ref[prev] = w
        z_ref[...] = jnp.broadcast_to(z, z_ref.shape)

    kh = n // 4

    @pl.when(p2 & (c >= 4))
    def _matmul():
        k0 = (c - 4) * kh
        wb = sc_ref[prev, :, pl.ds(k0, kh)].astype(jnp.bfloat16)
        xh = xbf_ref[prev, pl.ds(k0, kh), :]
        part = jax.lax.dot_general(
            wb, xh, (((1,), (0,)), ((), ())),
            preferred_element_type=jnp.float32)

        @pl.when(c == 4)
        def _():
            o_ref[0] = part

        @pl.when((c > 4) & (c < 7))
        def _():
            o_ref[0] = o_ref[0] + part

        @pl.when(c == 7)
        def _():
            o_ref[0] = (o_ref[0] + part) / z_ref[:, :1]


def _sortable(scores):
    bits = jax.lax.bitcast_convert_type(scores, jnp.int32)
    return bits ^ ((bits >> 31) & jnp.int32(0x7FFFFFFF))


def kernel(input, seed):
    B, N, H = input.shape
    S = seed.shape[1]
    assert N % (_N_CHUNKS * 128) == 0
    chunk = N // _N_CHUNKS
    last = _N_CHUNKS - 1
    body = functools.partial(
        _body, topk=min(_TOPK, N), scale=_N_HEADS ** -0.5, chunk=chunk,
        n_batches=B)
    return pl.pallas_call(
        body,
        grid=(B + 1, _N_CHUNKS),
        in_specs=[
            pl.BlockSpec(
                (1, chunk, H),
                lambda vb, c: (jnp.minimum(vb, B - 1),
                               jnp.where(vb >= B, last, c), 0)),
            pl.BlockSpec((1, S, H), lambda vb, c: (0, 0, 0)),
        ],
        out_specs=pl.BlockSpec(
            (1, S, H), lambda vb, c: (jnp.maximum(vb - 1, 0), 0, 0)),
        out_shape=jax.ShapeDtypeStruct((B, S, H), jnp.float32),
        scratch_shapes=[
            pltpu.VMEM((2, N, H), jnp.bfloat16),
            pltpu.VMEM((2, S, N), jnp.float32),
            pltpu.VMEM((S, 128), jnp.int32),
            pltpu.VMEM((S, 128), jnp.float32),
        ],
    )(input, seed)


# speedup vs baseline: 4.2120x; 1.0932x over previous
import functools

import jax
import jax.numpy as jnp
from jax.experimental import pallas as pl
from jax.experimental.pallas import tpu as pltpu

_TOPK = 128
_N_HEADS = 12
_N_CHUNKS = 8
_LOOKAHEAD = 3


def _radix_span(sortable, prefix, topk, lo, hi):
    def bit_step(i, pfx):
        bit = jnp.left_shift(jnp.int32(1), jnp.int32(30) - i)
        cand = pfx + bit
        cnt = jnp.sum((sortable >= cand).astype(jnp.int32), axis=1,
                      keepdims=True)
        return jnp.where(cnt >= topk, cand, pfx)

    return jax.lax.fori_loop(lo, hi, bit_step, prefix)


def _sortable(scores):
    bits = jax.lax.bitcast_convert_type(scores, jnp.int32)
    return bits ^ ((bits >> 31) & jnp.int32(0x7FFFFFFF))


def _body(x_hbm, q_ref, o_ref, slab_ref, sc_ref, st_ref, z_ref, sems, *,
          topk, scale, chunk, n_batches):
    vb = pl.program_id(0)
    c = pl.program_id(1)
    cur = vb % 2
    prev = (vb + 1) % 2
    g = vb * _N_CHUNKS + c
    total = n_batches * _N_CHUNKS

    def chunk_copy(g2):
        b2 = g2 // _N_CHUNKS
        c2 = g2 % _N_CHUNKS
        return pltpu.make_async_copy(
            x_hbm.at[b2, pl.ds(c2 * chunk, chunk), :],
            slab_ref.at[b2 % 2, pl.ds(c2 * chunk, chunk), :],
            sems.at[g2 % (_LOOKAHEAD + 1)])

    @pl.when(g == 0)
    def _prologue():
        for i in range(_LOOKAHEAD):
            chunk_copy(jnp.int32(i)).start()

    @pl.when(g + _LOOKAHEAD < total)
    def _issue():
        chunk_copy(g + _LOOKAHEAD).start()

    @pl.when(vb < n_batches)
    def _chunk():
        chunk_copy(g).wait()
        xc = slab_ref[cur, pl.ds(c * chunk, chunk), :]
        q = q_ref[0]
        sc_ref[cur, :, pl.ds(c * chunk, chunk)] = jax.lax.dot_general(
            q.astype(jnp.bfloat16), xc.astype(jnp.bfloat16),
            (((1,), (1,)), ((), ())),
            preferred_element_type=jnp.float32)

    p2 = vb > 0
    n = sc_ref.shape[2]

    @pl.when(p2 & (c == 0))
    def _radix0():
        sortable = _sortable(sc_ref[prev])
        n_nonneg = jnp.sum((sortable >= 0).astype(jnp.int32), axis=1,
                           keepdims=True)
        prefix = jnp.where(n_nonneg >= topk, jnp.int32(0),
                           jnp.int32(-(2 ** 31)))
        prefix = _radix_span(sortable, prefix, topk, 0, 10)
        st_ref[...] = jnp.broadcast_to(prefix, st_ref.shape)

    @pl.when(p2 & (c == 1))
    def _radix1():
        sortable = _sortable(sc_ref[prev])
        prefix = _radix_span(sortable, st_ref[:, :1], topk, 10, 20)
        st_ref[...] = jnp.broadcast_to(prefix, st_ref.shape)

    @pl.when(p2 & (c == 2))
    def _radix2():
        sortable = _sortable(sc_ref[prev])
        prefix = _radix_span(sortable, st_ref[:, :1], topk, 20, 31)
        st_ref[...] = jnp.broadcast_to(prefix, st_ref.shape)

    @pl.when(p2 & (c == 3))
    def _weights():
        scores = sc_ref[prev]
        sortable = _sortable(scores)
        mask = sortable >= st_ref[:, :1]
        rmax = jnp.max(scores, axis=1, keepdims=True)
        w = jnp.where(mask, jnp.exp((scores - rmax) * scale), 0.0)
        z = jnp.sum(w, axis=1, keepdims=True)
        sc_ref[prev] = w
        z_ref[...] = jnp.broadcast_to(z, z_ref.shape)

    kh = n // 4
    sub = max(kh // 4, 8)

    @pl.when(p2 & (c >= 4))
    def _matmul():
        k0 = (c - 4) * kh

        def sub_mm(i, acc):
            ks = k0 + i * sub
            wb = sc_ref[prev, :, pl.ds(ks, sub)].astype(jnp.bfloat16)
            xh = slab_ref[prev, pl.ds(ks, sub), :].astype(jnp.bfloat16)
            return acc + jax.lax.dot_general(
                wb, xh, (((1,), (0,)), ((), ())),
                preferred_element_type=jnp.float32)

        part = jax.lax.fori_loop(
            0, kh // sub, sub_mm,
            jnp.zeros((sc_ref.shape[1], slab_ref.shape[2]), jnp.float32))

        @pl.when(c == 4)
        def _():
            o_ref[0] = part

        @pl.when((c > 4) & (c < 7))
        def _():
            o_ref[0] = o_ref[0] + part

        @pl.when(c == 7)
        def _():
            o_ref[0] = (o_ref[0] + part) / z_ref[:, :1]


def kernel(input, seed):
    B, N, H = input.shape
    S = seed.shape[1]
    assert N % (_N_CHUNKS * 128) == 0
    chunk = N // _N_CHUNKS
    body = functools.partial(
        _body, topk=min(_TOPK, N), scale=_N_HEADS ** -0.5, chunk=chunk,
        n_batches=B)
    return pl.pallas_call(
        body,
        grid=(B + 1, _N_CHUNKS),
        in_specs=[
            pl.BlockSpec(memory_space=pltpu.MemorySpace.HBM),
            pl.BlockSpec((1, S, H), lambda vb, c: (0, 0, 0)),
        ],
        out_specs=pl.BlockSpec(
            (1, S, H), lambda vb, c: (jnp.maximum(vb - 1, 0), 0, 0)),
        out_shape=jax.ShapeDtypeStruct((B, S, H), jnp.float32),
        scratch_shapes=[
            pltpu.VMEM((2, N, H), jnp.float32),
            pltpu.VMEM((2, S, N), jnp.float32),
            pltpu.VMEM((S, 128), jnp.int32),
            pltpu.VMEM((S, 128), jnp.float32),
            pltpu.SemaphoreType.DMA((_LOOKAHEAD + 1,)),
        ],
    )(input, seed)


# speedup vs baseline: 5.4550x; 1.2951x over previous
import functools

import jax
import jax.numpy as jnp
from jax.experimental import pallas as pl
from jax.experimental.pallas import tpu as pltpu

_TOPK = 128
_N_HEADS = 12
_N_CHUNKS = 8
_LOOKAHEAD = 3


def _count_ge(sortable, cand):
    return jnp.sum((sortable >= cand).astype(jnp.int32), axis=1,
                   keepdims=True)


def _radix_rounds(sortable, prefix, topk, lo, hi):
    def round_step(i, pfx):
        bhi = jnp.left_shift(jnp.int32(1), jnp.int32(30) - 2 * i)
        blo = jnp.left_shift(jnp.int32(1), jnp.int32(29) - 2 * i)
        c1 = pfx + bhi
        c2 = pfx + bhi + blo
        c0 = pfx + blo
        n1 = _count_ge(sortable, c1)
        n2 = _count_ge(sortable, c2)
        n0 = _count_ge(sortable, c0)
        return jnp.where(n1 >= topk,
                         jnp.where(n2 >= topk, c2, c1),
                         jnp.where(n0 >= topk, c0, pfx))

    return jax.lax.fori_loop(lo, hi, round_step, prefix)


def _sortable(scores):
    bits = jax.lax.bitcast_convert_type(scores, jnp.int32)
    return bits ^ ((bits >> 31) & jnp.int32(0x7FFFFFFF))


def _hunks(n):
    base = n // (5 * 128) * 128
    sizes = [base] * 5
    left = (n - 5 * base) // 128
    for i in range(left):
        sizes[i] += 128
    offs, o = [], 0
    for s in sizes:
        offs.append(o)
        o += s
    return list(zip(offs, sizes))


def _body(x_hbm, q_ref, o_ref, slab_ref, sc_ref, st_ref, z_ref, sems, *,
          topk, scale, chunk, n_batches):
    vb = pl.program_id(0)
    c = pl.program_id(1)
    cur = vb % 2
    prev = (vb + 1) % 2
    g = vb * _N_CHUNKS + c
    total = n_batches * _N_CHUNKS

    def chunk_copy(g2):
        b2 = g2 // _N_CHUNKS
        c2 = g2 % _N_CHUNKS
        return pltpu.make_async_copy(
            x_hbm.at[b2, pl.ds(c2 * chunk, chunk), :],
            slab_ref.at[b2 % 2, pl.ds(c2 * chunk, chunk), :],
            sems.at[g2 % (_LOOKAHEAD + 1)])

    @pl.when(g == 0)
    def _prologue():
        for i in range(_LOOKAHEAD):
            chunk_copy(jnp.int32(i)).start()

    @pl.when(g + _LOOKAHEAD < total)
    def _issue():
        chunk_copy(g + _LOOKAHEAD).start()

    @pl.when(vb < n_batches)
    def _chunk():
        chunk_copy(g).wait()
        xc = slab_ref[cur, pl.ds(c * chunk, chunk), :]
        q = q_ref[0]
        sc_ref[cur, :, pl.ds(c * chunk, chunk)] = jax.lax.dot_general(
            q.astype(jnp.bfloat16), xc.astype(jnp.bfloat16),
            (((1,), (1,)), ((), ())),
            preferred_element_type=jnp.float32)

    p2 = vb > 0
    n = sc_ref.shape[2]

    @pl.when(p2 & (c == 0))
    def _radix0():
        sortable = _sortable(sc_ref[prev])
        n_nonneg = _count_ge(sortable, jnp.int32(0))
        prefix = jnp.where(n_nonneg >= topk, jnp.int32(0),
                           jnp.int32(-(2 ** 31)))
        prefix = _radix_rounds(sortable, prefix, topk, 0, 8)
        st_ref[...] = jnp.broadcast_to(prefix, st_ref.shape)

    @pl.when(p2 & (c == 1))
    def _radix1():
        sortable = _sortable(sc_ref[prev])
        prefix = _radix_rounds(sortable, st_ref[:, :1], topk, 8, 15)
        cand = prefix + jnp.int32(1)
        prefix = jnp.where(_count_ge(sortable, cand) >= topk, cand, prefix)
        st_ref[...] = jnp.broadcast_to(prefix, st_ref.shape)

    @pl.when(p2 & (c == 2))
    def _weights():
        scores = sc_ref[prev]
        sortable = _sortable(scores)
        mask = sortable >= st_ref[:, :1]
        rmax = jnp.max(scores, axis=1, keepdims=True)
        w = jnp.where(mask, jnp.exp((scores - rmax) * scale), 0.0)
        z = jnp.sum(w, axis=1, keepdims=True)
        sc_ref[prev] = w
        z_ref[...] = jnp.broadcast_to(z, z_ref.shape)

    for j, (k0, kh) in enumerate(_hunks(n)):
        @pl.when(p2 & (c == 3 + j))
        def _matmul(k0=k0, kh=kh, j=j):
            sub = kh // 4
            parts = []
            for i in range(4):
                ks = k0 + i * sub
                wb = sc_ref[prev, :, pl.ds(ks, sub)].astype(jnp.bfloat16)
                xh = slab_ref[prev, pl.ds(ks, sub), :].astype(jnp.bfloat16)
                parts.append(jax.lax.dot_general(
                    wb, xh, (((1,), (0,)), ((), ())),
                    preferred_element_type=jnp.float32))
            part = (parts[0] + parts[1]) + (parts[2] + parts[3])

            if j == 0:
                o_ref[0] = part
            elif j < 4:
                o_ref[0] = o_ref[0] + part
            else:
                o_ref[0] = (o_ref[0] + part) / z_ref[:, :1]


def kernel(input, seed):
    B, N, H = input.shape
    S = seed.shape[1]
    assert N % (_N_CHUNKS * 128) == 0
    chunk = N // _N_CHUNKS
    body = functools.partial(
        _body, topk=min(_TOPK, N), scale=_N_HEADS ** -0.5, chunk=chunk,
        n_batches=B)
    return pl.pallas_call(
        body,
        grid=(B + 1, _N_CHUNKS),
        in_specs=[
            pl.BlockSpec(memory_space=pltpu.MemorySpace.HBM),
            pl.BlockSpec((1, S, H), lambda vb, c: (0, 0, 0)),
        ],
        out_specs=pl.BlockSpec(
            (1, S, H), lambda vb, c: (jnp.maximum(vb - 1, 0), 0, 0)),
        out_shape=jax.ShapeDtypeStruct((B, S, H), jnp.float32),
        scratch_shapes=[
            pltpu.VMEM((2, N, H), jnp.float32),
            pltpu.VMEM((2, S, N), jnp.float32),
            pltpu.VMEM((S, 128), jnp.int32),
            pltpu.VMEM((S, 128), jnp.float32),
            pltpu.SemaphoreType.DMA((_LOOKAHEAD + 1,)),
        ],
    )(input, seed)
